# baseline (speedup 1.0000x reference)
import functools

import jax
import jax.numpy as jnp
from jax import lax
from jax.experimental import pallas as pl
from jax.experimental.pallas import tpu as pltpu
from jax.experimental.pallas import tpu_sc as plsc

N = 1024
E = 32768
D_IN = 128
F1, F2, F3 = 128, 64, 32
K = 16
T = 16

_NS = 16
_ROWS = N // _NS
_CHUNK = 16384
_LANES = 16

_f32 = jnp.float32

def _sc_body(edges, adj, acc, srcb, dstb):
    c = lax.axis_index("c")
    s = lax.axis_index("s")
    base = s * _ROWS
    zeros16 = jnp.zeros((_LANES,), _f32)
    ones = jnp.ones((_LANES,), _f32)

    def zbody(j, carry):
        acc[j >> 6, pl.ds((j & 63) * _LANES, _LANES)] = zeros16
        return carry

    lax.fori_loop(0, _ROWS * N // _LANES, zbody, 0, unroll=16)

    for t in range(E // _CHUNK):
        off = t * _CHUNK
        pltpu.sync_copy(edges.at[c, 0, pl.ds(off, _CHUNK)], srcb)
        pltpu.sync_copy(edges.at[c, 1, pl.ds(off, _CHUNK)], dstb)

        def step(i, carry2):
            src16 = srcb[pl.ds(i * _LANES, _LANES)]
            dst16 = dstb[pl.ds(i * _LANES, _LANES)]
            rel = dst16 - base
            mask = plsc.bitcast(rel, jnp.uint32) < jnp.uint32(_ROWS)
            col = jnp.where(mask, src16, 0)
            plsc.addupdate_scatter(acc, [rel, col], ones, mask=mask)
            return carry2

        lax.fori_loop(0, _CHUNK // _LANES, step, 0, unroll=8)

    pltpu.sync_copy(acc, adj.at[c, pl.ds(base, _ROWS)])


@functools.cache
def _sc_build_adj():
    mesh = plsc.VectorSubcoreMesh(core_axis_name="c", subcore_axis_name="s")
    return pl.kernel(
        _sc_body,
        mesh=mesh,
        compiler_params=pltpu.CompilerParams(needs_layout_passes=False),
        out_type=jax.ShapeDtypeStruct((2, N, N), _f32),
        scratch_types=[
            pltpu.VMEM((_ROWS, N), _f32),
            pltpu.VMEM((_CHUNK,), jnp.int32),
            pltpu.VMEM((_CHUNK,), jnp.int32),
        ],
    )


def _mm(a, b, prec=None):
    return jnp.dot(a, b, preferred_element_type=_f32, precision=prec)


def _mm_nt(a, b, prec=None):
    return lax.dot_general(a, b, (((1,), (1,)), ((), ())),
                           preferred_element_type=_f32, precision=prec)


def _mm_tn(a, b, prec=None):
    return lax.dot_general(a, b, (((0,), (0,)), ((), ())),
                           preferred_element_type=_f32, precision=prec)


_HI = lax.Precision.HIGHEST


def _sigmoid(x):
    return 1.0 / (1.0 + jnp.exp(-x))


def _rsqrt(x):
    y = lax.rsqrt(x)
    return y * (1.5 - 0.5 * x * y * y)


def _gcn_body(adj_ref, x_ref, W1r, b1r, W2r, b2r, W3r, b3r, af_ref):
    rows = lax.broadcasted_iota(jnp.int32, (N, N), 0)
    cols = lax.broadcasted_iota(jnp.int32, (N, N), 1)
    eye = jnp.where(rows == cols, 1.0, 0.0)
    ones_row = jnp.ones((1, N), _f32)

    A = adj_ref[0]
    deg_col = jnp.sum(A, axis=1, keepdims=True) + 1.0
    deg_row = _mm_nt(ones_row, A, _HI) + 1.0
    dinv_col = _rsqrt(deg_col)
    dinv_row = _rsqrt(deg_row)
    S = (A + eye) * (dinv_col * dinv_row)
    h = jnp.maximum(_mm(S, _mm(x_ref[0], W1r[...]), _HI) + b1r[...], 0.0)
    h = jnp.maximum(_mm(S, _mm(h, W2r[...]), _HI) + b2r[...], 0.0)
    af_ref[0] = _mm(S, _mm(h, W3r[...]), _HI) + b3r[...]


def _gcn_call(adj, feats, W1, b1r, W2, b2r, W3, b3r):
    full = lambda s: pl.BlockSpec(s, lambda g: (0,) * len(s))
    return pl.pallas_call(
        _gcn_body,
        grid=(2,),
        out_shape=jax.ShapeDtypeStruct((2, N, F3), _f32),
        in_specs=[
            pl.BlockSpec((1, N, N), lambda g: (g, 0, 0)),
            pl.BlockSpec((1, N, D_IN), lambda g: (g, 0, 0)),
            full((D_IN, F1)), full((1, F1)),
            full((F1, F2)), full((1, F2)),
            full((F2, F3)), full((1, F3)),
        ],
        out_specs=pl.BlockSpec((1, N, F3), lambda g: (g, 0, 0)),
    )(adj, feats, W1, b1r, W2, b2r, W3, b3r)


_TILE = 256
_NT = N // _TILE


def _score_body(vc_s, vm_s, bs_s, af_ref,
                Wcr, Wmr, War, Wttr, btr, WtbAr, WtbBr,
                Wf1r, bf1r, Wf2r, bf2r, Wf3r, bf3r, Wsr,
                mapm_ref, score_ref, acc_ref):
    i = pl.program_id(0)
    row0 = i * _TILE
    af1t = af_ref[0, pl.ds(row0, _TILE), :]
    af2 = af_ref[1]

    cost = jnp.zeros((_TILE, N), _f32)
    mapm = jnp.zeros((_TILE, N), _f32)
    bf = jnp.bfloat16
    for k in range(K):
        mc = _mm_nt(_mm(af1t, Wcr[k]), af2)
        cost = cost + vc_s[0, k] * jnp.maximum(mc, 0.0)
        mk = _mm_nt(_mm(af1t, Wmr[k]), af2)
        rk = jnp.maximum(mk, 0.0).astype(bf).astype(_f32)
        vk = vm_s[0, k].astype(bf).astype(_f32)
        mapm = mapm + vk * rk
    mapm_ref[...] = mapm

    rowmax = jnp.max(mapm, axis=1, keepdims=True)
    ex = jnp.exp(mapm - rowmax)
    soft = ex / jnp.sum(ex, axis=1, keepdims=True)
    tot_col = jnp.sum(soft * cost, axis=1, keepdims=True)
    partial = jnp.sum(tot_col)
    prev = jnp.where(i == 0, 0.0, acc_ref[0, 0])
    total = prev + partial
    acc_ref[0, 0] = total

    @pl.when(i == _NT - 1)
    def _tail():
        af1 = af_ref[0]

        def attention(af):
            g = jnp.tanh(jnp.mean(_mm(af, War[...]), axis=0, keepdims=True))
            sgate = _sigmoid(_mm_nt(af, g))
            return _mm_tn(sgate, af)

        p1 = attention(af1)
        p2 = attention(af2)

        G = _mm_tn(p1, p2)
        colid = lax.broadcasted_iota(jnp.int32, (1, T), 1)
        sc_row = jnp.zeros((1, T), _f32)
        for t in range(T):
            st = jnp.sum(G * Wttr[t])
            sc_row = sc_row + jnp.where(colid == t, st, 0.0)
        blk_row = _mm(p1, WtbAr[...]) + _mm(p2, WtbBr[...])
        sv = jnp.maximum(sc_row + blk_row + btr[...], 0.0)
        sv = jnp.maximum(_mm(sv, Wf1r[...]) + bf1r[...], 0.0)
        sv = jnp.maximum(_mm(sv, Wf2r[...]) + bf2r[...], 0.0)
        sv = jnp.maximum(_mm(sv, Wf3r[...]) + bf3r[...], 0.0)
        bias11 = _mm(sv, Wsr[...]) + bs_s[0, 0]
        score_ref[...] = _sigmoid(total + bias11)


def _score_call(vc2, vm2, bs2, af, *dense_ops):
    smem = pl.BlockSpec(memory_space=pltpu.SMEM)
    full = pl.BlockSpec(memory_space=pltpu.VMEM)
    return pl.pallas_call(
        _score_body,
        grid=(_NT,),
        out_shape=(jax.ShapeDtypeStruct((N, N), _f32),
                   jax.ShapeDtypeStruct((1, 1), _f32)),
        in_specs=[smem, smem, smem, full] + [full] * len(dense_ops),
        out_specs=(pl.BlockSpec((_TILE, N), lambda i: (i, 0)),
                   pl.BlockSpec((1, 1), lambda i: (0, 0))),
        scratch_shapes=[pltpu.SMEM((1, 1), _f32)],
    )(vc2, vm2, bs2, af, *dense_ops)


def kernel(features_1, features_2, edge_index_1, edge_index_2, A_1, A_2,
           mapping, W1, b1, W2, b2, W3, b3, Wc, vc, Wm, vm, Wa, Wt, Wtb, bt,
           Wf1, bf1, Wf2, bf2, Wf3, bf3, Ws, bs):
    edges = jnp.stack([edge_index_1.astype(jnp.int32),
                       edge_index_2.astype(jnp.int32)])
    adj = _sc_build_adj()(edges)
    feats = jnp.stack([features_1, features_2])

    af = _gcn_call(adj, feats,
                   W1, b1.reshape(1, F1), W2, b2.reshape(1, F2),
                   W3, b3.reshape(1, F3))

    mapm, score11 = _score_call(
        vc.reshape(1, K), vm.reshape(1, K), bs.reshape(1, 1),
        af,
        Wc, Wm, Wa,
        jnp.transpose(Wt, (2, 0, 1)),
        bt.reshape(1, T),
        jnp.transpose(Wtb[:, :F3]),
        jnp.transpose(Wtb[:, F3:]),
        Wf1, bf1.reshape(1, -1), Wf2, bf2.reshape(1, -1),
        Wf3, bf3.reshape(1, -1), Ws)
    return (mapm, score11.reshape(-1))
